# baseline (speedup 1.0000x reference)
import functools

import jax
import jax.numpy as jnp
from jax import lax
from jax.experimental import pallas as pl
from jax.experimental.pallas import tpu as pltpu
from jax.experimental.pallas import tpu_sc as plsc

HIDDEN = 2048
NUM_EXPERTS = 16
TOP_K = 2
BLOCK = 1024
N_TOKENS = 16384
NW = 32
TPW = N_TOKENS // NW
NGROUPS = TPW // 16


def _tc_body(x_ref, w_ref, logits_ref):
    logits_ref[...] = lax.dot_general(
        w_ref[...], x_ref[...],
        dimension_numbers=(((1,), (1,)), ((), ())),
        preferred_element_type=jnp.float32,
    )


def _tc_logits(x, W):
    n, h = x.shape
    grid = (n // BLOCK,)
    return pl.pallas_call(
        _tc_body,
        grid=grid,
        in_specs=[
            pl.BlockSpec((BLOCK, h), lambda i: (i, 0)),
            pl.BlockSpec((NUM_EXPERTS, h), lambda i: (0, 0)),
        ],
        out_specs=pl.BlockSpec((NUM_EXPERTS, BLOCK), lambda i: (0, i)),
        out_shape=jax.ShapeDtypeStruct((NUM_EXPERTS, n), jnp.float32),
        compiler_params=pltpu.CompilerParams(
            dimension_semantics=("arbitrary",),
        ),
    )(x, W)


def _sc_route_body(lg_hbm, idx_hbm, w_hbm, lg_v, i1_v, i2_v, w1_v, w2_v):
    wid = lax.axis_index("s") * 2 + lax.axis_index("c")
    base = wid * TPW
    pltpu.sync_copy(lg_hbm.at[:, pl.ds(base, TPW)], lg_v)

    def group(g, carry):
        neg = jnp.full((16,), -jnp.inf, jnp.float32)
        zero = jnp.zeros((16,), jnp.int32)
        m1, m2, i1, i2 = neg, neg, zero, zero
        for e in range(NUM_EXPERTS):
            v = lg_v[e, pl.ds(g * 16, 16)]
            ev = jnp.full((16,), e, jnp.int32)
            gt1 = v > m1
            vgt2 = v > m2
            m2 = jnp.where(gt1, m1, jnp.where(vgt2, v, m2))
            i2 = jnp.where(gt1, i1, jnp.where(vgt2, ev, i2))
            m1 = jnp.where(gt1, v, m1)
            i1 = jnp.where(gt1, ev, i1)
        ex = jnp.exp(m2 - m1)
        w1 = 1.0 / (1.0 + ex)
        w2 = 1.0 - w1
        sl = pl.ds(g * 16, 16)
        i1_v[sl] = i1
        i2_v[sl] = i2
        w1_v[sl] = w1
        w2_v[sl] = w2
        return carry

    lax.fori_loop(0, NGROUPS, group, 0)
    pltpu.sync_copy(i1_v, idx_hbm.at[0, pl.ds(base, TPW)])
    pltpu.sync_copy(i2_v, idx_hbm.at[1, pl.ds(base, TPW)])
    pltpu.sync_copy(w1_v, w_hbm.at[0, pl.ds(base, TPW)])
    pltpu.sync_copy(w2_v, w_hbm.at[1, pl.ds(base, TPW)])


def _sc_route(logits_t):
    mesh = plsc.VectorSubcoreMesh(core_axis_name="c", subcore_axis_name="s")
    k = functools.partial(
        pl.kernel,
        mesh=mesh,
        out_type=[
            jax.ShapeDtypeStruct((TOP_K, N_TOKENS), jnp.int32),
            jax.ShapeDtypeStruct((TOP_K, N_TOKENS), jnp.float32),
        ],
        scratch_types=[
            pltpu.VMEM((NUM_EXPERTS, TPW), jnp.float32),
            pltpu.VMEM((TPW,), jnp.int32),
            pltpu.VMEM((TPW,), jnp.int32),
            pltpu.VMEM((TPW,), jnp.float32),
            pltpu.VMEM((TPW,), jnp.float32),
        ],
    )(_sc_route_body)
    return k(logits_t)


def kernel(hidden_states, W):
    b, s, h = hidden_states.shape
    x = hidden_states.reshape(-1, h)
    logits_t = _tc_logits(x, W)
    idx_t, w_t = _sc_route(logits_t)
    return logits_t.T, idx_t.T, w_t.T


# speedup vs baseline: 1.4522x; 1.4522x over previous
import jax
import jax.numpy as jnp
from jax import lax
from jax.experimental import pallas as pl
from jax.experimental.pallas import tpu as pltpu

HIDDEN = 2048
NUM_EXPERTS = 16
TOP_K = 2
BLOCK = 1024


def _body(x_ref, w_ref, logits_ref, idx_ref, w_out_ref):
    logits = lax.dot_general(
        w_ref[...], x_ref[...],
        dimension_numbers=(((1,), (1,)), ((), ())),
        preferred_element_type=jnp.float32,
    )
    b = logits.shape[1]
    iota = lax.broadcasted_iota(jnp.int32, (NUM_EXPERTS, b), 0)
    m1 = jnp.max(logits, axis=0, keepdims=True)
    idx1 = jnp.min(jnp.where(logits == m1, iota, NUM_EXPERTS), axis=0, keepdims=True)
    masked = jnp.where(iota == idx1, -jnp.inf, logits)
    m2 = jnp.max(masked, axis=0, keepdims=True)
    idx2 = jnp.min(jnp.where(masked == m2, iota, NUM_EXPERTS), axis=0, keepdims=True)
    e = jnp.exp(m2 - m1)
    w1 = 1.0 / (1.0 + e)
    w2 = 1.0 - w1
    logits_ref[...] = logits
    row = lax.broadcasted_iota(jnp.int32, (TOP_K, b), 0)
    idx_ref[...] = jnp.where(row == 0, idx1, idx2)
    w_out_ref[...] = jnp.where(row == 0, w1, w2)


def kernel(hidden_states, W):
    b, s, h = hidden_states.shape
    x = hidden_states.reshape(-1, h)
    n = x.shape[0]
    grid = (n // BLOCK,)
    logits_t, idx_t, w_t = pl.pallas_call(
        _body,
        grid=grid,
        in_specs=[
            pl.BlockSpec((BLOCK, h), lambda i: (i, 0)),
            pl.BlockSpec((NUM_EXPERTS, h), lambda i: (0, 0)),
        ],
        out_specs=[
            pl.BlockSpec((NUM_EXPERTS, BLOCK), lambda i: (0, i)),
            pl.BlockSpec((TOP_K, BLOCK), lambda i: (0, i)),
            pl.BlockSpec((TOP_K, BLOCK), lambda i: (0, i)),
        ],
        out_shape=[
            jax.ShapeDtypeStruct((NUM_EXPERTS, n), jnp.float32),
            jax.ShapeDtypeStruct((TOP_K, n), jnp.int32),
            jax.ShapeDtypeStruct((TOP_K, n), jnp.float32),
        ],
        compiler_params=pltpu.CompilerParams(
            dimension_semantics=("arbitrary",),
        ),
    )(x, W)
    return logits_t.T, idx_t.T, w_t.T
